# Initial kernel scaffold; baseline (speedup 1.0000x reference)
import jax
import jax.numpy as jnp
from jax.experimental import pallas as pl


def kernel(x, edge_index, edge_attr, params):
    raise NotImplementedError("write your pallas kernel here")


# speedup vs baseline: 2.3862x; 2.3862x over previous
import functools

import jax
import jax.numpy as jnp
from jax import lax
from jax.experimental import pallas as pl
from jax.experimental.pallas import tpu as pltpu
import jax.experimental.pallas.tpu_sc as plsc

N = 10000
E = 320000
NC = 2
NS = 16
NW = NC * NS
EPW = E // NW
C = 80
CH = EPW // C
RPS = N // NS
CW = 16
BE = 2000
GRID = E // BE

_mesh = plsc.VectorSubcoreMesh(core_axis_name="c", subcore_axis_name="s",
                               num_cores=NC, num_subcores=NS)


def _wid():
    return lax.axis_index("s") * NC + lax.axis_index("c")


@functools.partial(jax.jit, static_argnums=(2,))
def _sc_gather1(table, idx3, w):
    @functools.partial(
        pl.kernel,
        out_type=jax.ShapeDtypeStruct((E, w), jnp.float32),
        mesh=_mesh,
        compiler_params=pltpu.CompilerParams(use_tc_tiling_on_sc=False),
        scratch_types=[
            pltpu.VMEM((CH, C), jnp.int32),
            pltpu.VMEM((C, w), jnp.float32),
            pltpu.SemaphoreType.DMA,
        ],
    )
    def k(table_h, idx_h, out_h, idx_v, rows_v, sem):
        wid = _wid()
        pltpu.sync_copy(idx_h.at[wid], idx_v)
        base = wid * EPW

        def chunk(j, carry):
            pltpu.async_copy(table_h.at[idx_v.at[j]], rows_v, sem).wait()
            pltpu.sync_copy(rows_v, out_h.at[pl.ds(base + j * C, C)])
            return carry

        lax.fori_loop(0, CH, chunk, 0)

    return k(table, idx3)


@functools.partial(jax.jit, static_argnums=(4,))
def _sc_gather2(ta, tb, idxa3, idxb3, w):
    @functools.partial(
        pl.kernel,
        out_type=(jax.ShapeDtypeStruct((E, w), jnp.float32),
                  jax.ShapeDtypeStruct((E, w), jnp.float32)),
        mesh=_mesh,
        compiler_params=pltpu.CompilerParams(use_tc_tiling_on_sc=False),
        scratch_types=[
            pltpu.VMEM((CH, C), jnp.int32),
            pltpu.VMEM((CH, C), jnp.int32),
            pltpu.VMEM((C, w), jnp.float32),
            pltpu.VMEM((C, w), jnp.float32),
            pltpu.SemaphoreType.DMA,
            pltpu.SemaphoreType.DMA,
        ],
    )
    def k(ta_h, tb_h, idxa_h, idxb_h, oa_h, ob_h,
          ia_v, ib_v, ra_v, rb_v, sa, sb):
        wid = _wid()
        pltpu.sync_copy(idxa_h.at[wid], ia_v)
        pltpu.sync_copy(idxb_h.at[wid], ib_v)
        base = wid * EPW

        def chunk(j, carry):
            ca = pltpu.async_copy(ta_h.at[ia_v.at[j]], ra_v, sa)
            cb = pltpu.async_copy(tb_h.at[ib_v.at[j]], rb_v, sb)
            ca.wait()
            pltpu.sync_copy(ra_v, oa_h.at[pl.ds(base + j * C, C)])
            cb.wait()
            pltpu.sync_copy(rb_v, ob_h.at[pl.ds(base + j * C, C)])
            return carry

        lax.fori_loop(0, CH, chunk, 0)

    return k(ta, tb, idxa3, idxb3)


@functools.partial(jax.jit, static_argnums=(3,))
def _sc_scatter(vals, idx3, zeros_w, w):
    @functools.partial(
        pl.kernel,
        out_type=jax.ShapeDtypeStruct((NC, N, w), jnp.float32),
        mesh=_mesh,
        compiler_params=pltpu.CompilerParams(use_tc_tiling_on_sc=False),
        scratch_types=[
            pltpu.VMEM((CH, C), jnp.int32),
            pltpu.VMEM((C, w), jnp.float32),
            pltpu.VMEM_SHARED((N, w), jnp.float32),
            pltpu.SemaphoreType.DMA,
        ],
    )
    def k(vals_h, idx_h, zw_h, out_h, idx_v, val_v, acc_s, sem):
        cid = lax.axis_index("c")
        sid = lax.axis_index("s")
        wid = sid * NC + cid
        row0 = sid * RPS
        pltpu.sync_copy(zw_h.at[pl.ds(row0, RPS)], acc_s.at[pl.ds(row0, RPS)])
        pltpu.sync_copy(idx_h.at[wid], idx_v)
        plsc.subcore_barrier()
        base = wid * EPW

        def chunk(j, carry):
            pltpu.sync_copy(vals_h.at[pl.ds(base + j * C, C)], val_v)
            pltpu.sync_copy(val_v, acc_s.at[idx_v.at[j]], add=True)
            return carry

        lax.fori_loop(0, CH, chunk, 0)
        plsc.subcore_barrier()
        pltpu.sync_copy(acc_s.at[pl.ds(row0, RPS)],
                        out_h.at[cid, pl.ds(row0, RPS)])

    return k(vals, idx3, zeros_w)


@functools.partial(jax.jit, static_argnums=(4,))
def _sc_scatter_cnt(vals, idx3, zeros_w, zeros_c, w):
    @functools.partial(
        pl.kernel,
        out_type=(jax.ShapeDtypeStruct((NC, N, w), jnp.float32),
                  jax.ShapeDtypeStruct((NC, N, CW), jnp.float32)),
        mesh=_mesh,
        compiler_params=pltpu.CompilerParams(use_tc_tiling_on_sc=False),
        scratch_types=[
            pltpu.VMEM((CH, C), jnp.int32),
            pltpu.VMEM((C, w), jnp.float32),
            pltpu.VMEM((C, CW), jnp.float32),
            pltpu.VMEM_SHARED((N, w), jnp.float32),
            pltpu.VMEM_SHARED((N, CW), jnp.float32),
            pltpu.SemaphoreType.DMA,
        ],
    )
    def k(vals_h, idx_h, zw_h, zc_h, out_h, outc_h,
          idx_v, val_v, ones_v, acc_s, accc_s, sem):
        cid = lax.axis_index("c")
        sid = lax.axis_index("s")
        wid = sid * NC + cid
        row0 = sid * RPS
        pltpu.sync_copy(zw_h.at[pl.ds(row0, RPS)], acc_s.at[pl.ds(row0, RPS)])
        pltpu.sync_copy(zc_h.at[pl.ds(row0, RPS)], accc_s.at[pl.ds(row0, RPS)])
        pltpu.sync_copy(idx_h.at[wid], idx_v)

        def fill(i, carry):
            ones_v[i, :] = jnp.full((CW,), 1.0, jnp.float32)
            return carry

        lax.fori_loop(0, C, fill, 0)
        plsc.subcore_barrier()
        base = wid * EPW

        def chunk(j, carry):
            pltpu.sync_copy(vals_h.at[pl.ds(base + j * C, C)], val_v)
            pltpu.sync_copy(val_v, acc_s.at[idx_v.at[j]], add=True)
            pltpu.sync_copy(ones_v, accc_s.at[idx_v.at[j]], add=True)
            return carry

        lax.fori_loop(0, CH, chunk, 0)
        plsc.subcore_barrier()
        pltpu.sync_copy(acc_s.at[pl.ds(row0, RPS)],
                        out_h.at[cid, pl.ds(row0, RPS)])
        pltpu.sync_copy(accc_s.at[pl.ds(row0, RPS)],
                        outc_h.at[cid, pl.ds(row0, RPS)])

    return k(vals, idx3, zeros_w, zeros_c)


def _full(shape):
    return pl.BlockSpec(shape, lambda i: (0, 0))


@functools.partial(jax.jit, static_argnums=(5,))
def _tc_enc_mlp(g, ea, w1bt, w2t, b2, w):
    def body(g_ref, ea_ref, w1b_ref, w2_ref, b2_ref, out_ref):
        r = jnp.maximum(g_ref[...] + ea_ref[...] @ w1b_ref[...], 0.0)
        out_ref[...] = jnp.maximum(r @ w2_ref[...] + b2_ref[...], 0.0)

    return pl.pallas_call(
        body,
        grid=(GRID,),
        in_specs=[
            pl.BlockSpec((BE, 32), lambda i: (i, 0)),
            pl.BlockSpec((BE, 16), lambda i: (i, 0)),
            _full((16, 32)),
            _full((32, w)),
            _full((1, w)),
        ],
        out_specs=pl.BlockSpec((BE, w), lambda i: (i, 0)),
        out_shape=jax.ShapeDtypeStruct((E, w), jnp.float32),
    )(g, ea, w1bt, w2t, b2)


@jax.jit
def _tc_dec0_head(zt, zs, wab, wb, b1, w2, b2,
                  wp, wq, b1f, w2f, b2f, w3f, b3f, sp, sq, bd):
    def body(zt_ref, zs_ref, wab_ref, wb_ref, b1_ref, w2_ref, b2_ref,
             wp_ref, wq_ref, b1f_ref, w2f_ref, b2f_ref, w3f_ref, b3f_ref,
             sp_ref, sq_ref, bd_ref, h_ref, pred_ref):
        ztv = zt_ref[...]
        zsv = zs_ref[...]
        r = jnp.maximum(ztv @ wab_ref[...] + zsv @ wb_ref[...] + b1_ref[...],
                        0.0)
        h_ref[...] = jnp.maximum(r @ w2_ref[...] + b2_ref[...], 0.0)
        s = zsv + ztv
        d = jnp.abs(zsv - ztv)
        u = jnp.maximum(s @ wp_ref[...] + d @ wq_ref[...] + b1f_ref[...], 0.0)
        uu = jnp.maximum(u @ w2f_ref[...] + b2f_ref[...] + u, 0.0)
        pred_ref[...] = (uu @ w3f_ref[...] + b3f_ref[...]
                         + s @ sp_ref[...] + d @ sq_ref[...] + bd_ref[...])

    return pl.pallas_call(
        body,
        grid=(GRID,),
        in_specs=[
            pl.BlockSpec((BE, 32), lambda i: (i, 0)),
            pl.BlockSpec((BE, 32), lambda i: (i, 0)),
            _full((32, 32)), _full((32, 32)), _full((1, 32)),
            _full((32, 64)), _full((1, 64)),
            _full((32, 32)), _full((32, 32)), _full((1, 32)),
            _full((32, 32)), _full((1, 32)),
            _full((32, 16)), _full((1, 16)),
            _full((32, 16)), _full((32, 16)), _full((1, 16)),
        ],
        out_specs=(pl.BlockSpec((BE, 64), lambda i: (i, 0)),
                   pl.BlockSpec((BE, 16), lambda i: (i, 0))),
        out_shape=(jax.ShapeDtypeStruct((E, 64), jnp.float32),
                   jax.ShapeDtypeStruct((E, 16), jnp.float32)),
    )(zt, zs, wab, wb, b1, w2, b2, wp, wq, b1f, w2f, b2f, w3f, b3f,
      sp, sq, bd)


@jax.jit
def _tc_relu_add(a, b):
    def body(a_ref, b_ref, o_ref):
        o_ref[...] = jnp.maximum(a_ref[...] + b_ref[...], 0.0)

    return pl.pallas_call(
        body,
        grid=(GRID,),
        in_specs=[pl.BlockSpec((BE, 32), lambda i: (i, 0)),
                  pl.BlockSpec((BE, 32), lambda i: (i, 0))],
        out_specs=pl.BlockSpec((BE, 32), lambda i: (i, 0)),
        out_shape=jax.ShapeDtypeStruct((E, 32), jnp.float32),
    )(a, b)


def kernel(x, edge_index, edge_attr, params):
    src = edge_index[0].astype(jnp.int32)
    dst = edge_index[1].astype(jnp.int32)
    dst3 = dst.reshape(NW, CH, C)
    src3 = src.reshape(NW, CH, C)
    f32 = jnp.float32
    z16 = jnp.zeros((N, CW), f32)
    z32 = jnp.zeros((N, 32), f32)
    z64 = jnp.zeros((N, 64), f32)
    z128 = jnp.zeros((N, 128), f32)

    def enc_edge_mlp(h, bp, w):
        din = h.shape[1]
        w1 = bp["l1"]["w"]
        t = h @ w1[:, :din].T + bp["l1"]["b"]
        g = _sc_gather1(t, dst3, 32)
        return _tc_enc_mlp(g, edge_attr, w1[:, din:].T,
                           bp["l2"]["w"].T, bp["l2"]["b"].reshape(1, w), w)

    bp = params["enc0"]
    hh = enc_edge_mlp(x, bp, 128)
    part, partc = _sc_scatter_cnt(hh, dst3, z128, z16, 128)
    cnt = (partc[0, :, :1] + partc[1, :, :1])
    cntc = jnp.maximum(cnt, 1.0)
    h1 = (part[0] + part[1]) / cntc + x

    bp = params["enc1"]
    hh = enc_edge_mlp(h1, bp, 64)
    part = _sc_scatter(hh, dst3, z64, 64)
    h2 = ((part[0] + part[1]) / cntc
          + h1 @ bp["res"]["w"].T + bp["res"]["b"])

    bp = params["enc2"]
    hh = enc_edge_mlp(h2, bp, 32)
    part = _sc_scatter(hh, dst3, z32, 32)
    z = ((part[0] + part[1]) / cntc
         + h2 @ bp["res"]["w"].T + bp["res"]["b"])

    zt, zs = _sc_gather2(z, z, dst3, src3, 32)
    bp = params["dec0"]
    w1 = bp["l1"]["w"]
    wa, wb = w1[:, :32], w1[:, 32:]
    f1 = params["fc1"]["w"]
    f1a, f1b = f1[:, :32], f1[:, 32:]
    fd = params["fc_direct"]["w"]
    fda, fdb = fd[:, :32], fd[:, 32:]
    h0, pred = _tc_dec0_head(
        zt, zs,
        (wa - wb).T, wb.T, bp["l1"]["b"].reshape(1, 32),
        bp["l2"]["w"].T, bp["l2"]["b"].reshape(1, 64),
        ((f1a + f1b) / 2).T, ((f1b - f1a) / 2).T,
        params["fc1"]["b"].reshape(1, 32),
        params["fc2"]["w"].T, params["fc2"]["b"].reshape(1, 32),
        params["fc3"]["w"].T, params["fc3"]["b"].reshape(1, 16),
        ((fda + fdb) / 2).T, ((fdb - fda) / 2).T,
        params["fc_direct"]["b"].reshape(1, 16))
    part = _sc_scatter(h0, dst3, z64, 64)
    h3 = ((part[0] + part[1]) / cntc
          + z @ bp["res"]["w"].T + bp["res"]["b"])

    bp = params["dec1"]
    w1 = bp["l1"]["w"]
    wa, wb = w1[:, :64], w1[:, 64:]
    ta = h3 @ (wa - wb).T + bp["l1"]["b"]
    tb = h3 @ wb.T
    ga, gb = _sc_gather2(ta, tb, dst3, src3, 32)
    r = _tc_relu_add(ga, gb)
    part = _sc_scatter(r, dst3, z32, 32)
    sr = part[0] + part[1]
    x_rec = ((sr @ bp["l2"]["w"].T + cnt * bp["l2"]["b"]) / cntc
             + h3 @ bp["res"]["w"].T + bp["res"]["b"])
    return x_rec, pred


# speedup vs baseline: 2.8750x; 1.2048x over previous
import functools

import jax
import jax.numpy as jnp
from jax import lax
from jax.experimental import pallas as pl
from jax.experimental.pallas import tpu as pltpu
import jax.experimental.pallas.tpu_sc as plsc

N = 10000
E = 320000
NC = 2
NS = 16
NW = NC * NS
EPW = E // NW
C = 80
CH = EPW // C
RPS = N // NS
CW = 16
BE = 4000
GRID = E // BE

_mesh = plsc.VectorSubcoreMesh(core_axis_name="c", subcore_axis_name="s",
                               num_cores=NC, num_subcores=NS)
_untiled = pltpu.CompilerParams(use_tc_tiling_on_sc=False)


def _wid():
    return lax.axis_index("s") * NC + lax.axis_index("c")


@functools.partial(jax.jit, static_argnums=(2,))
def _sc_gather1(table, idx3, w):
    @functools.partial(
        pl.kernel,
        out_type=jax.ShapeDtypeStruct((E, w), jnp.float32),
        mesh=_mesh,
        compiler_params=_untiled,
        scratch_types=[
            pltpu.VMEM((CH, C), jnp.int32),
            pltpu.VMEM((C, w), jnp.float32),
            pltpu.SemaphoreType.DMA,
        ],
    )
    def k(table_h, idx_h, out_h, idx_v, rows_v, sem):
        wid = _wid()
        pltpu.sync_copy(idx_h.at[wid], idx_v)
        base = wid * EPW

        def chunk(j, carry):
            pltpu.async_copy(table_h.at[idx_v.at[j]], rows_v, sem).wait()
            pltpu.sync_copy(rows_v, out_h.at[pl.ds(base + j * C, C)])
            return carry

        lax.fori_loop(0, CH, chunk, 0)

    return k(table, idx3)


@jax.jit
def _sc_gather2(ta, tb, idxa3, idxb3):
    @functools.partial(
        pl.kernel,
        out_type=(jax.ShapeDtypeStruct((E, 64), jnp.float32),
                  jax.ShapeDtypeStruct((E, 64), jnp.float32)),
        mesh=_mesh,
        compiler_params=_untiled,
        scratch_types=[
            pltpu.VMEM((CH, C), jnp.int32),
            pltpu.VMEM((CH, C), jnp.int32),
            pltpu.VMEM((C, 64), jnp.float32),
            pltpu.VMEM((C, 64), jnp.float32),
            pltpu.SemaphoreType.DMA,
            pltpu.SemaphoreType.DMA,
        ],
    )
    def k(ta_h, tb_h, idxa_h, idxb_h, oa_h, ob_h,
          ia_v, ib_v, ra_v, rb_v, sa, sb):
        wid = _wid()
        pltpu.sync_copy(idxa_h.at[wid], ia_v)
        pltpu.sync_copy(idxb_h.at[wid], ib_v)
        base = wid * EPW

        def chunk(j, carry):
            ca = pltpu.async_copy(ta_h.at[ia_v.at[j]], ra_v, sa)
            cb = pltpu.async_copy(tb_h.at[ib_v.at[j]], rb_v, sb)
            ca.wait()
            pltpu.sync_copy(ra_v, oa_h.at[pl.ds(base + j * C, C)])
            cb.wait()
            pltpu.sync_copy(rb_v, ob_h.at[pl.ds(base + j * C, C)])
            return carry

        lax.fori_loop(0, CH, chunk, 0)

    return k(ta, tb, idxa3, idxb3)


@functools.partial(jax.jit, static_argnums=(3,))
def _sc_scatter(vals, idx3, zeros_w, w):
    @functools.partial(
        pl.kernel,
        out_type=jax.ShapeDtypeStruct((NC, N, w), jnp.float32),
        mesh=_mesh,
        compiler_params=_untiled,
        scratch_types=[
            pltpu.VMEM((CH, C), jnp.int32),
            pltpu.VMEM((C, w), jnp.float32),
            pltpu.VMEM_SHARED((N, w), jnp.float32),
            pltpu.SemaphoreType.DMA,
        ],
    )
    def k(vals_h, idx_h, zw_h, out_h, idx_v, val_v, acc_s, sem):
        cid = lax.axis_index("c")
        sid = lax.axis_index("s")
        wid = sid * NC + cid
        row0 = sid * RPS
        pltpu.sync_copy(zw_h.at[pl.ds(row0, RPS)], acc_s.at[pl.ds(row0, RPS)])
        pltpu.sync_copy(idx_h.at[wid], idx_v)
        plsc.subcore_barrier()
        base = wid * EPW

        def chunk(j, carry):
            pltpu.sync_copy(vals_h.at[pl.ds(base + j * C, C)], val_v)
            pltpu.sync_copy(val_v, acc_s.at[idx_v.at[j]], add=True)
            return carry

        lax.fori_loop(0, CH, chunk, 0)
        plsc.subcore_barrier()
        pltpu.sync_copy(acc_s.at[pl.ds(row0, RPS)],
                        out_h.at[cid, pl.ds(row0, RPS)])

    return k(vals, idx3, zeros_w)


@jax.jit
def _sc_scatter_cnt(vals, idx3, zeros_w, zeros_c):
    w = 128

    @functools.partial(
        pl.kernel,
        out_type=(jax.ShapeDtypeStruct((NC, N, w), jnp.float32),
                  jax.ShapeDtypeStruct((NC, N, CW), jnp.float32)),
        mesh=_mesh,
        compiler_params=_untiled,
        scratch_types=[
            pltpu.VMEM((CH, C), jnp.int32),
            pltpu.VMEM((C, w), jnp.float32),
            pltpu.VMEM((C, CW), jnp.float32),
            pltpu.VMEM_SHARED((N, w), jnp.float32),
            pltpu.VMEM_SHARED((N, CW), jnp.float32),
            pltpu.SemaphoreType.DMA,
        ],
    )
    def k(vals_h, idx_h, zw_h, zc_h, out_h, outc_h,
          idx_v, val_v, ones_v, acc_s, accc_s, sem):
        cid = lax.axis_index("c")
        sid = lax.axis_index("s")
        wid = sid * NC + cid
        row0 = sid * RPS
        pltpu.sync_copy(zw_h.at[pl.ds(row0, RPS)], acc_s.at[pl.ds(row0, RPS)])
        pltpu.sync_copy(zc_h.at[pl.ds(row0, RPS)], accc_s.at[pl.ds(row0, RPS)])
        pltpu.sync_copy(idx_h.at[wid], idx_v)

        def fill(i, carry):
            ones_v[i, :] = jnp.full((CW,), 1.0, jnp.float32)
            return carry

        lax.fori_loop(0, C, fill, 0)
        plsc.subcore_barrier()
        base = wid * EPW

        def chunk(j, carry):
            pltpu.sync_copy(vals_h.at[pl.ds(base + j * C, C)], val_v)
            pltpu.sync_copy(val_v, acc_s.at[idx_v.at[j]], add=True)
            pltpu.sync_copy(ones_v, accc_s.at[idx_v.at[j]], add=True)
            return carry

        lax.fori_loop(0, CH, chunk, 0)
        plsc.subcore_barrier()
        pltpu.sync_copy(acc_s.at[pl.ds(row0, RPS)],
                        out_h.at[cid, pl.ds(row0, RPS)])
        pltpu.sync_copy(accc_s.at[pl.ds(row0, RPS)],
                        outc_h.at[cid, pl.ds(row0, RPS)])

    return k(vals, idx3, zeros_w, zeros_c)


@jax.jit
def _sc_dec1_fused(tab, idxd3, idxs3, zeros_w):
    @functools.partial(
        pl.kernel,
        out_type=jax.ShapeDtypeStruct((NC, N, 32), jnp.float32),
        mesh=_mesh,
        compiler_params=_untiled,
        scratch_types=[
            pltpu.VMEM((CH, C), jnp.int32),
            pltpu.VMEM((CH, C), jnp.int32),
            pltpu.VMEM((C, 64), jnp.float32),
            pltpu.VMEM((C, 64), jnp.float32),
            pltpu.VMEM((C, 32), jnp.float32),
            pltpu.VMEM_SHARED((N, 32), jnp.float32),
            pltpu.SemaphoreType.DMA,
            pltpu.SemaphoreType.DMA,
        ],
    )
    def k(tab_h, idxd_h, idxs_h, zw_h, out_h,
          id_v, is_v, a_v, b_v, r_v, acc_s, sa, sb):
        cid = lax.axis_index("c")
        sid = lax.axis_index("s")
        wid = sid * NC + cid
        row0 = sid * RPS
        pltpu.sync_copy(zw_h.at[pl.ds(row0, RPS)], acc_s.at[pl.ds(row0, RPS)])
        pltpu.sync_copy(idxd_h.at[wid], id_v)
        pltpu.sync_copy(idxs_h.at[wid], is_v)
        plsc.subcore_barrier()

        def chunk(j, carry):
            ca = pltpu.async_copy(tab_h.at[id_v.at[j]], a_v, sa)
            cb = pltpu.async_copy(tab_h.at[is_v.at[j]], b_v, sb)
            ca.wait()
            cb.wait()

            def row(i, carry2):
                r_v[i, pl.ds(0, 16)] = jnp.maximum(
                    a_v[i, pl.ds(0, 16)] + b_v[i, pl.ds(32, 16)], 0.0)
                r_v[i, pl.ds(16, 16)] = jnp.maximum(
                    a_v[i, pl.ds(16, 16)] + b_v[i, pl.ds(48, 16)], 0.0)
                return carry2

            lax.fori_loop(0, C, row, 0)
            pltpu.sync_copy(r_v, acc_s.at[id_v.at[j]], add=True)
            return carry

        lax.fori_loop(0, CH, chunk, 0)
        plsc.subcore_barrier()
        pltpu.sync_copy(acc_s.at[pl.ds(row0, RPS)],
                        out_h.at[cid, pl.ds(row0, RPS)])

    return k(tab, idxd3, idxs3, zeros_w)


def _full(shape):
    return pl.BlockSpec(shape, lambda i: (0, 0))


@functools.partial(jax.jit, static_argnums=(5,))
def _tc_enc_mlp(g, ea, w1bt, w2t, b2, w):
    def body(g_ref, ea_ref, w1b_ref, w2_ref, b2_ref, out_ref):
        r = jnp.maximum(g_ref[...] + ea_ref[...] @ w1b_ref[...], 0.0)
        out_ref[...] = jnp.maximum(r @ w2_ref[...] + b2_ref[...], 0.0)

    return pl.pallas_call(
        body,
        grid=(GRID,),
        in_specs=[
            pl.BlockSpec((BE, 32), lambda i: (i, 0)),
            pl.BlockSpec((BE, 16), lambda i: (i, 0)),
            _full((16, 32)),
            _full((32, w)),
            _full((1, w)),
        ],
        out_specs=pl.BlockSpec((BE, w), lambda i: (i, 0)),
        out_shape=jax.ShapeDtypeStruct((E, w), jnp.float32),
    )(g, ea, w1bt, w2t, b2)


@jax.jit
def _tc_dec0_head(gd, gs, w2, b2,
                  wp, wq, b1f, w2f, b2f, w3f, b3f, sp, sq, bd):
    def body(gd_ref, gs_ref, w2_ref, b2_ref,
             wp_ref, wq_ref, b1f_ref, w2f_ref, b2f_ref, w3f_ref, b3f_ref,
             sp_ref, sq_ref, bd_ref, h_ref, pred_ref):
        gd_v = gd_ref[...]
        gs_v = gs_ref[...]
        pre = gd_v + gs_v
        r = jnp.maximum(pre[:, :32], 0.0)
        h_ref[...] = jnp.maximum(r @ w2_ref[...] + b2_ref[...], 0.0)
        s = pre[:, 32:]
        d = jnp.abs(gd_v[:, 32:] - gs_v[:, 32:])
        u = jnp.maximum(s @ wp_ref[...] + d @ wq_ref[...] + b1f_ref[...], 0.0)
        uu = jnp.maximum(u @ w2f_ref[...] + b2f_ref[...] + u, 0.0)
        pred_ref[...] = (uu @ w3f_ref[...] + b3f_ref[...]
                         + s @ sp_ref[...] + d @ sq_ref[...] + bd_ref[...])

    return pl.pallas_call(
        body,
        grid=(GRID,),
        in_specs=[
            pl.BlockSpec((BE, 64), lambda i: (i, 0)),
            pl.BlockSpec((BE, 64), lambda i: (i, 0)),
            _full((32, 64)), _full((1, 64)),
            _full((32, 32)), _full((32, 32)), _full((1, 32)),
            _full((32, 32)), _full((1, 32)),
            _full((32, 16)), _full((1, 16)),
            _full((32, 16)), _full((32, 16)), _full((1, 16)),
        ],
        out_specs=(pl.BlockSpec((BE, 64), lambda i: (i, 0)),
                   pl.BlockSpec((BE, 16), lambda i: (i, 0))),
        out_shape=(jax.ShapeDtypeStruct((E, 64), jnp.float32),
                   jax.ShapeDtypeStruct((E, 16), jnp.float32)),
    )(gd, gs, w2, b2, wp, wq, b1f, w2f, b2f, w3f, b3f, sp, sq, bd)


def kernel(x, edge_index, edge_attr, params):
    src = edge_index[0].astype(jnp.int32)
    dst = edge_index[1].astype(jnp.int32)
    dst3 = dst.reshape(NW, CH, C)
    src3 = src.reshape(NW, CH, C)
    f32 = jnp.float32
    z16 = jnp.zeros((N, CW), f32)
    z32 = jnp.zeros((N, 32), f32)
    z64 = jnp.zeros((N, 64), f32)
    z128 = jnp.zeros((N, 128), f32)

    def enc_edge_mlp(h, bp, w):
        din = h.shape[1]
        w1 = bp["l1"]["w"]
        t = h @ w1[:, :din].T + bp["l1"]["b"]
        g = _sc_gather1(t, dst3, 32)
        return _tc_enc_mlp(g, edge_attr, w1[:, din:].T,
                           bp["l2"]["w"].T, bp["l2"]["b"].reshape(1, w), w)

    bp = params["enc0"]
    hh = enc_edge_mlp(x, bp, 128)
    part, partc = _sc_scatter_cnt(hh, dst3, z128, z16)
    cnt = (partc[0, :, :1] + partc[1, :, :1])
    cntc = jnp.maximum(cnt, 1.0)
    h1 = (part[0] + part[1]) / cntc + x

    bp = params["enc1"]
    hh = enc_edge_mlp(h1, bp, 64)
    part = _sc_scatter(hh, dst3, z64, 64)
    h2 = ((part[0] + part[1]) / cntc
          + h1 @ bp["res"]["w"].T + bp["res"]["b"])

    bp = params["enc2"]
    hh = enc_edge_mlp(h2, bp, 32)
    part = _sc_scatter(hh, dst3, z32, 32)
    z = ((part[0] + part[1]) / cntc
         + h2 @ bp["res"]["w"].T + bp["res"]["b"])

    bp = params["dec0"]
    w1 = bp["l1"]["w"]
    wa, wb = w1[:, :32], w1[:, 32:]
    eye = jnp.eye(32, dtype=f32)
    md = jnp.concatenate([wa - wb, eye], axis=0)
    ms = jnp.concatenate([wb, eye], axis=0)
    bd_pad = jnp.concatenate([bp["l1"]["b"], jnp.zeros((32,), f32)])
    tab_d = z @ md.T + bd_pad
    tab_s = z @ ms.T
    gd, gs = _sc_gather2(tab_d, tab_s, dst3, src3)
    f1 = params["fc1"]["w"]
    f1a, f1b = f1[:, :32], f1[:, 32:]
    fd = params["fc_direct"]["w"]
    fda, fdb = fd[:, :32], fd[:, 32:]
    h0, pred = _tc_dec0_head(
        gd, gs,
        bp["l2"]["w"].T, bp["l2"]["b"].reshape(1, 64),
        ((f1a + f1b) / 2).T, ((f1b - f1a) / 2).T,
        params["fc1"]["b"].reshape(1, 32),
        params["fc2"]["w"].T, params["fc2"]["b"].reshape(1, 32),
        params["fc3"]["w"].T, params["fc3"]["b"].reshape(1, 16),
        ((fda + fdb) / 2).T, ((fdb - fda) / 2).T,
        params["fc_direct"]["b"].reshape(1, 16))
    part = _sc_scatter(h0, dst3, z64, 64)
    h3 = ((part[0] + part[1]) / cntc
          + z @ bp["res"]["w"].T + bp["res"]["b"])

    bp = params["dec1"]
    w1 = bp["l1"]["w"]
    wa, wb = w1[:, :64], w1[:, 64:]
    m1 = jnp.concatenate([wa - wb, wb], axis=0)
    b1_pad = jnp.concatenate([bp["l1"]["b"], jnp.zeros((32,), f32)])
    tab = h3 @ m1.T + b1_pad
    part = _sc_dec1_fused(tab, dst3, src3, z32)
    sr = part[0] + part[1]
    x_rec = ((sr @ bp["l2"]["w"].T + cnt * bp["l2"]["b"]) / cntc
             + h3 @ bp["res"]["w"].T + bp["res"]["b"])
    return x_rec, pred
